# baseline (speedup 1.0000x reference)
import functools

import jax
import jax.numpy as jnp
from jax import lax
from jax.experimental import pallas as pl
from jax.experimental.pallas import tpu as pltpu
from jax.experimental.pallas import tpu_sc as plsc

_NC, _NS = 2, 16
_NW = _NC * _NS
B_SC = 512


def _sc_add_range(x, emb, b_lo):
    B, T, D = x.shape
    per_w = B_SC // _NW
    half = per_w // 2
    mesh = plsc.VectorSubcoreMesh(core_axis_name="c", subcore_axis_name="s")

    @functools.partial(
        pl.kernel,
        mesh=mesh,
        out_type=jax.ShapeDtypeStruct((B_SC, T, D), jnp.float32),
        scratch_types=[
            pltpu.VMEM((T, D), jnp.float32),
            pltpu.VMEM((T, D), jnp.float32),
            pltpu.VMEM((T, D), jnp.float32),
            pltpu.SemaphoreType.DMA,
            pltpu.SemaphoreType.DMA,
            pltpu.SemaphoreType.DMA,
            pltpu.SemaphoreType.DMA,
        ],
    )
    def k(x_hbm, emb_hbm, out_hbm, emb_v, buf0, buf1, si0, si1, so0, so1):
        wid = lax.axis_index("s") * _NC + lax.axis_index("c")
        lbase = wid * per_w
        gbase = b_lo + lbase
        pltpu.sync_copy(emb_hbm, emb_v)

        def add_table(buf):
            @plsc.parallel_loop(0, T, unroll=2)
            def _(r):
                for j in range(D // 16):
                    sl = pl.ds(j * 16, 16)
                    buf[r, sl] = buf[r, sl] + emb_v[r, sl]

        pltpu.async_copy(x_hbm.at[gbase], buf0, si0)
        pltpu.async_copy(x_hbm.at[gbase + 1], buf1, si1)

        def body(i, carry):
            g0 = gbase + 2 * i
            l0 = lbase + 2 * i
            pltpu.make_async_copy(x_hbm.at[g0], buf0, si0).wait()
            add_table(buf0)
            pltpu.async_copy(buf0, out_hbm.at[l0], so0)
            pltpu.make_async_copy(x_hbm.at[g0 + 1], buf1, si1).wait()
            add_table(buf1)
            pltpu.async_copy(buf1, out_hbm.at[l0 + 1], so1)

            @pl.when(i < half - 1)
            def _():
                pltpu.make_async_copy(buf0, out_hbm.at[l0], so0).wait()
                pltpu.async_copy(x_hbm.at[g0 + 2], buf0, si0)
                pltpu.make_async_copy(buf1, out_hbm.at[l0 + 1], so1).wait()
                pltpu.async_copy(x_hbm.at[g0 + 3], buf1, si1)

            return carry

        lax.fori_loop(0, half, body, 0)
        pltpu.make_async_copy(buf0, out_hbm.at[lbase + per_w - 2], so0).wait()
        pltpu.make_async_copy(buf1, out_hbm.at[lbase + per_w - 1], so1).wait()

    return k(x, emb)


def _add_body(x_ref, emb_ref, o_ref):
    o_ref[...] = x_ref[...] + emb_ref[...][None, :, :]


def _tc_add_range(x, emb, b_hi):
    B, T, D = x.shape
    B_BLK = 128
    return pl.pallas_call(
        _add_body,
        grid=(b_hi // B_BLK,),
        in_specs=[
            pl.BlockSpec((B_BLK, T, D), lambda i: (i, 0, 0)),
            pl.BlockSpec((T, D), lambda i: (0, 0)),
        ],
        out_specs=pl.BlockSpec((B_BLK, T, D), lambda i: (i, 0, 0)),
        out_shape=jax.ShapeDtypeStruct((b_hi, T, D), x.dtype),
    )(x, emb)


def kernel(x, emb_table):
    B, T, D = x.shape
    emb = emb_table[:T]
    b_tc = B - B_SC
    sc_out = _sc_add_range(x, emb, b_tc)
    tc_out = _tc_add_range(x, emb, b_tc)
    return jnp.concatenate([tc_out, sc_out], axis=0)


# speedup vs baseline: 1.9485x; 1.9485x over previous
import functools

import jax
import jax.numpy as jnp
from jax import lax
from jax.experimental import pallas as pl
from jax.experimental.pallas import tpu as pltpu
from jax.experimental.pallas import tpu_sc as plsc

_NC, _NS = 2, 16
_NW = _NC * _NS


def _sc_gather(emb_table, T):
    D = emb_table.shape[1]
    rows = 16
    n_w = (T + rows - 1) // rows
    mesh = plsc.VectorSubcoreMesh(
        core_axis_name="c", subcore_axis_name="s", num_cores=1
    )

    @functools.partial(
        pl.kernel,
        mesh=mesh,
        out_type=jax.ShapeDtypeStruct((T, D), jnp.float32),
        scratch_types=[
            pltpu.VMEM((rows, D), jnp.float32),
            pltpu.SemaphoreType.DMA,
        ],
    )
    def k(emb_hbm, out_hbm, rows_v, sem):
        wid = lax.axis_index("s")

        @pl.when(wid < n_w)
        def _():
            base = jnp.minimum(wid * rows, T - rows)
            idx = lax.iota(jnp.int32, rows) + base
            pltpu.async_copy(emb_hbm.at[idx], rows_v, sem).wait()
            pltpu.sync_copy(rows_v, out_hbm.at[pl.ds(base, rows)])

    return k(emb_table)


def _add_body(x_ref, emb_ref, o_ref):
    o_ref[...] = x_ref[...] + emb_ref[...][None, :, :]


def _tc_add(x, pos_emb):
    B, T, D = x.shape
    B_BLK = 128
    return pl.pallas_call(
        _add_body,
        grid=(B // B_BLK,),
        in_specs=[
            pl.BlockSpec((B_BLK, T, D), lambda i: (i, 0, 0)),
            pl.BlockSpec((T, D), lambda i: (0, 0)),
        ],
        out_specs=pl.BlockSpec((B_BLK, T, D), lambda i: (i, 0, 0)),
        out_shape=jax.ShapeDtypeStruct((B, T, D), x.dtype),
        compiler_params=pltpu.CompilerParams(
            dimension_semantics=("parallel",)
        ),
    )(x, pos_emb)


def kernel(x, emb_table):
    T = x.shape[1]
    pos_emb = _sc_gather(emb_table, T)
    return _tc_add(x, pos_emb)
